# baseline (speedup 1.0000x reference)
import jax
import jax.numpy as jnp
from jax import lax
from jax.experimental import pallas as pl
from jax.experimental.pallas import tpu as pltpu
from jax.experimental.pallas import tpu_sc as plsc

_N = 100000
_E = 1600000
_NF = 8
_EF = 4
_NV = 100000
_EV = 1000
_D = 32
_H = 64

_NC = 2
_NS = 16
_HALF = _N // _NC
_LROWS = 50176
_NROW = _LROWS // 128
_NCHUNKS = _NROW // 8

_ECH_ROWS = 8
_ECHUNKS = 98
_TEROW = _ECH_ROWS * _ECHUNKS
_EROW = _TEROW * _NS
_EPAD = _EROW * 128

_POOL = 2176
_FLUSH = 1024


def _sc_body(nflat, eflat, nidx, eidx, dst, out,
             acc, nfidx, eidxv, dstv, pf0, pf1, pf2, pf3, psidx, sidx_dma,
             rows, semg, sems, semi, seme):
    pf = [pf0, pf1, pf2, pf3]
    c = lax.axis_index("c")
    s = lax.axis_index("s")
    cbase = c * _HALF
    il16 = lax.iota(jnp.int32, 16)

    def node_chunk(k, carry):
        cid = s + k * _NS

        @pl.when(cid < _NCHUNKS)
        def _do():
            rowb = cid * 8

            def fill_ids(par, q):
                for m in range(8):
                    sidx_dma[par, pl.ds(m * 16, 16)] = (
                        cid * 1024 + q * 128 + m * 16) + il16

            for f in range(_NF):
                pltpu.sync_copy(nidx.at[c, f, pl.ds(rowb, 8)], nfidx)
                fill_ids(0, 0)
                gd = [None] * 8
                sd = [None] * 8
                gd[0] = pltpu.async_copy(nflat.at[nfidx.at[0]], rows.at[0],
                                         semg)
                for q in range(8):
                    b = q % 2
                    gd[q].wait()
                    sd[q] = pltpu.async_copy(rows.at[b],
                                             acc.at[sidx_dma.at[b]], sems,
                                             add=(f > 0))
                    if q >= 1:
                        sd[q - 1].wait()
                    if q + 1 < 8:
                        fill_ids(1 - b, q + 1)
                        gd[q + 1] = pltpu.async_copy(nflat.at[nfidx.at[q + 1]],
                                                     rows.at[1 - b], semg)
                sd[7].wait()
        return carry

    lax.fori_loop(0, (_NCHUNKS + _NS - 1) // _NS, node_chunk, 0)
    plsc.subcore_barrier()

    ebase = s * _TEROW

    def flush_window(fbase):
        def fill_sidx(par, k):
            for m in range(8):
                sidx_dma[par, pl.ds(m * 16, 16)] = psidx[
                    pl.ds(pl.multiple_of(fbase + k * 128 + m * 16, 8), 16)]

        def gref(p):
            f, k = p % _EF, p // _EF
            off = pl.multiple_of(fbase + k * 128, 8)
            return eflat.at[pf[f].at[pl.ds(off, 128)]]

        nstep = 8 * _EF
        fill_sidx(0, 0)
        gd = [None] * nstep
        sd = [None] * nstep
        gd[0] = pltpu.async_copy(gref(0), rows.at[0], semg)
        for p in range(nstep):
            b = p % 2
            k = p // _EF
            gd[p].wait()
            sd[p] = pltpu.async_copy(rows.at[b], acc.at[sidx_dma.at[k % 2]],
                                     sems, add=True)
            if p >= 1:
                sd[p - 1].wait()
            if p + 1 < nstep:
                if (p + 1) % _EF == 0:
                    fill_sidx((p + 1) // _EF % 2, (p + 1) // _EF)
                gd[p + 1] = pltpu.async_copy(gref(p + 1), rows.at[1 - b], semg)
        sd[nstep - 1].wait()

    def compact_and_flush(rb, par, cnt, prefetch_rb):
        ecp = pltpu.async_copy(eidx.at[pl.ds(rb, _ECH_ROWS)], eidxv, seme)
        pltpu.make_async_copy(dst.at[pl.ds(rb, _ECH_ROWS)], dstv.at[par],
                              semi).wait()

        @pl.when(prefetch_rb < ebase + _TEROW)
        def _pref():
            pltpu.async_copy(dst.at[pl.ds(prefetch_rb, _ECH_ROWS)],
                             dstv.at[1 - par], semi)

        ecp.wait()
        for r in range(_ECH_ROWS):
            for j in range(8):
                v = dstv[par, r, pl.ds(j * 16, 16)]
                loc = v - cbase
                ok = (loc >= 0) & (loc < _HALF)
                oki = ok.astype(jnp.int32)
                pos = cnt + plsc.cumsum(oki) - 1
                plsc.store_scatter(psidx, [pos], loc, mask=ok)
                rfull = jnp.full((16,), r, jnp.int32)
                for f in range(_EF):
                    ev = plsc.load_gather(
                        eidxv, [rfull, il16 * 4 + (j * 64 + f)])
                    plsc.store_scatter(pf[f], [pos], ev + f * _EV, mask=ok)
                cnt = cnt + jnp.sum(oki)
        pad8 = (-cnt) & 7
        padmask = il16 < pad8
        padpos = cnt + il16
        plsc.store_scatter(psidx, [padpos], _HALF + (il16 & 127), mask=padmask)
        for f in range(_EF):
            plsc.store_scatter(pf[f], [padpos], jnp.zeros((16,), jnp.int32),
                               mask=padmask)
        cnt = cnt + pad8

        @pl.when(cnt >= _FLUSH)
        def _fl():
            flush_window(cnt - _FLUSH)

        return jnp.where(cnt >= _FLUSH, cnt - _FLUSH, cnt)

    pltpu.async_copy(dst.at[pl.ds(ebase, _ECH_ROWS)], dstv.at[0], semi)

    def edge_pair(i, cnt):
        rb = ebase + i * 2 * _ECH_ROWS
        cnt = compact_and_flush(rb, 0, cnt, rb + _ECH_ROWS)
        cnt = compact_and_flush(rb + _ECH_ROWS, 1, cnt, rb + 2 * _ECH_ROWS)
        return cnt

    cnt = lax.fori_loop(0, _ECHUNKS // 2, edge_pair, jnp.int32(0))

    def pad16(t, cnt):
        padn = (-cnt) & 127

        @pl.when(t * 16 < padn)
        def _p():
            m = il16 < (padn - t * 16)
            ppos = cnt + t * 16 + il16
            plsc.store_scatter(psidx, [ppos], _HALF + (il16 & 127), mask=m)
            for f in range(_EF):
                plsc.store_scatter(pf[f], [ppos],
                                   jnp.zeros((16,), jnp.int32), mask=m)
        return cnt

    lax.fori_loop(0, 8, pad16, cnt)
    ngroups = (cnt + 127) // 128

    def drain_group(k, _):
        @pl.when(k < ngroups)
        def _dg():
            for m in range(8):
                sidx_dma[0, pl.ds(m * 16, 16)] = psidx[
                    pl.ds(pl.multiple_of(k * 128 + m * 16, 8), 16)]
            gd = pltpu.async_copy(
                eflat.at[pf[0].at[pl.ds(pl.multiple_of(k * 128, 8), 128)]],
                rows.at[0], semg)
            for f in range(_EF):
                gd.wait()
                if f + 1 < _EF:
                    gd = pltpu.async_copy(
                        eflat.at[pf[f + 1].at[
                            pl.ds(pl.multiple_of(k * 128, 8), 128)]],
                        rows.at[(f + 1) % 2], semg)
                pltpu.sync_copy(rows.at[f % 2], acc.at[sidx_dma.at[0]],
                                add=True)
        return _

    lax.fori_loop(0, 8, drain_group, 0)
    plsc.subcore_barrier()

    def out_chunk(k, _):
        cid = s + k * _NS

        @pl.when(cid < _NCHUNKS)
        def _do():
            pltpu.sync_copy(acc.at[pl.ds(cid * 1024, 1024)],
                            out.at[c, pl.ds(cid * 1024, 1024)])
        return _

    lax.fori_loop(0, (_NCHUNKS + _NS - 1) // _NS, out_chunk, 0)


def _mlp_body(a_ref, we_ref, wd_ref, o_ref):
    a = a_ref[0]
    h = jnp.maximum(jnp.dot(a, we_ref[...], preferred_element_type=jnp.float32), 0.0)
    o_ref[...] = jnp.dot(h, wd_ref[...], preferred_element_type=jnp.float32)


@jax.jit
def kernel(x, edge_attr, edge_index, node_tables, edge_tables, W_enc, W_dec):
    nflat = node_tables.reshape(_NF * _NV, _D)
    eflat = edge_tables.reshape(_EF * _EV, _D)

    nidx = x + (jnp.arange(_NF, dtype=jnp.int32) * _NV)[None, :]
    padn = jnp.zeros((_LROWS - _HALF, _NF), jnp.int32)
    nidx = jnp.concatenate([nidx[:_HALF], padn, nidx[_HALF:], padn], axis=0)
    nidx = nidx.reshape(_NC, _LROWS, _NF).transpose(0, 2, 1)
    nidx = nidx.reshape(_NC, _NF, _NROW, 128)

    pade = jnp.zeros((_EPAD - _E, _EF), jnp.int32)
    eidx = jnp.concatenate([edge_attr, pade], axis=0).reshape(_EROW, _EF * 128)

    dst = jnp.concatenate([edge_index[1], jnp.full((_EPAD - _E,), -1, jnp.int32)])
    dst = dst.reshape(_EROW, 128)

    mesh = plsc.VectorSubcoreMesh(core_axis_name="c", subcore_axis_name="s",
                                  num_cores=_NC, num_subcores=_NS)
    agg = pl.kernel(
        _sc_body,
        out_type=jax.ShapeDtypeStruct((_NC, _LROWS, _D), jnp.float32),
        mesh=mesh,
        compiler_params=pltpu.CompilerParams(use_tc_tiling_on_sc=False,
                                             needs_layout_passes=False),
        scratch_types=[
            pltpu.VMEM_SHARED((_LROWS, _D), jnp.float32),
            pltpu.VMEM((8, 128), jnp.int32),
            pltpu.VMEM((_ECH_ROWS, _EF * 128), jnp.int32),
            pltpu.VMEM((2, _ECH_ROWS, 128), jnp.int32),
            pltpu.VMEM((_POOL,), jnp.int32),
            pltpu.VMEM((_POOL,), jnp.int32),
            pltpu.VMEM((_POOL,), jnp.int32),
            pltpu.VMEM((_POOL,), jnp.int32),
            pltpu.VMEM((_POOL,), jnp.int32),
            pltpu.VMEM((2, 128), jnp.int32),
            pltpu.VMEM((2, 128, _D), jnp.float32),
            pltpu.SemaphoreType.DMA,
            pltpu.SemaphoreType.DMA,
            pltpu.SemaphoreType.DMA,
            pltpu.SemaphoreType.DMA,
        ],
    )(nflat, eflat, nidx, eidx, dst)

    bm = 2000
    nb = _HALF // bm
    out = pl.pallas_call(
        _mlp_body,
        grid=(_NC, nb),
        in_specs=[
            pl.BlockSpec((1, bm, _D), lambda c, i: (c, i, 0)),
            pl.BlockSpec((_D, _H), lambda c, i: (0, 0)),
            pl.BlockSpec((_H, _D), lambda c, i: (0, 0)),
        ],
        out_specs=pl.BlockSpec((bm, _D), lambda c, i: (c * nb + i, 0)),
        out_shape=jax.ShapeDtypeStruct((_N, _D), jnp.float32),
    )(agg, W_enc, W_dec)
    return out


# speedup vs baseline: 2.0961x; 2.0961x over previous
import jax
import jax.numpy as jnp
from jax import lax
from jax.experimental import pallas as pl
from jax.experimental.pallas import tpu as pltpu
from jax.experimental.pallas import tpu_sc as plsc

_N = 100000
_E = 1600000
_NF = 8
_EF = 4
_NV = 100000
_EV = 1000
_D = 32
_H = 64

_NC = 2
_NS = 16
_HALF = _N // _NC
_LROWS = 50176
_NROW = _LROWS // 128
_NCHUNKS = _NROW // 8

_ECH_ROWS = 8
_ECHUNKS = 98
_TEROW = _ECH_ROWS * _ECHUNKS
_EROW = _TEROW * _NS
_EPAD = _EROW * 128

_POOL = 2176
_FLUSH = 1024


def _sc_body(nflat, eflat, nidx, eidx, dst, out,
             acc, nfidx, eidxv, dstv, pf0, pf1, pf2, pf3, psidx, sidx_dma,
             rows, semg, sems, semi, seme):
    pf = [pf0, pf1, pf2, pf3]
    c = lax.axis_index("c")
    s = lax.axis_index("s")
    cbase = c * _HALF
    il16 = lax.iota(jnp.int32, 16)

    def node_chunk(k, carry):
        cid = s + k * _NS

        @pl.when(cid < _NCHUNKS)
        def _do():
            rowb = cid * 8

            def fill_ids(par, q):
                for m in range(8):
                    sidx_dma[par, pl.ds(m * 16, 16)] = (
                        cid * 1024 + q * 128 + m * 16) + il16

            for f in range(_NF):
                pltpu.sync_copy(nidx.at[c, f, pl.ds(rowb, 8)], nfidx)
                fill_ids(0, 0)
                gd = [None] * 8
                sd = [None] * 8
                gd[0] = pltpu.async_copy(nflat.at[nfidx.at[0]], rows.at[0],
                                         semg)
                for q in range(8):
                    b = q % 2
                    gd[q].wait()
                    sd[q] = pltpu.async_copy(rows.at[b],
                                             acc.at[sidx_dma.at[b]], sems,
                                             add=(f > 0))
                    if q >= 1:
                        sd[q - 1].wait()
                    if q + 1 < 8:
                        fill_ids(1 - b, q + 1)
                        gd[q + 1] = pltpu.async_copy(nflat.at[nfidx.at[q + 1]],
                                                     rows.at[1 - b], semg)
                sd[7].wait()
        return carry

    lax.fori_loop(0, (_NCHUNKS + _NS - 1) // _NS, node_chunk, 0)
    plsc.subcore_barrier()

    ebase = s * _TEROW

    def flush_window(fbase):
        def fill_sidx(par, k):
            for m in range(8):
                sidx_dma[par, pl.ds(m * 16, 16)] = psidx[
                    pl.ds(pl.multiple_of(fbase + k * 128 + m * 16, 8), 16)]

        def gref(p):
            f, k = p % _EF, p // _EF
            off = pl.multiple_of(fbase + k * 128, 8)
            return eflat.at[pf[f].at[pl.ds(off, 128)]]

        nstep = 8 * _EF
        fill_sidx(0, 0)
        gd = [None] * nstep
        sd = [None] * nstep
        gd[0] = pltpu.async_copy(gref(0), rows.at[0], semg)
        for p in range(nstep):
            b = p % 2
            k = p // _EF
            gd[p].wait()
            sd[p] = pltpu.async_copy(rows.at[b], acc.at[sidx_dma.at[k % 2]],
                                     sems, add=True)
            if p >= 1:
                sd[p - 1].wait()
            if p + 1 < nstep:
                if (p + 1) % _EF == 0:
                    fill_sidx((p + 1) // _EF % 2, (p + 1) // _EF)
                gd[p + 1] = pltpu.async_copy(gref(p + 1), rows.at[1 - b], semg)
        sd[nstep - 1].wait()

    def compact_and_flush(rb, par, cnt, prefetch_rb):
        ecps = [pltpu.async_copy(eidx.at[f, pl.ds(rb, _ECH_ROWS)],
                                 eidxv.at[f], seme) for f in range(_EF)]
        pltpu.make_async_copy(dst.at[pl.ds(rb, _ECH_ROWS)], dstv.at[par],
                              semi).wait()

        @pl.when(prefetch_rb < ebase + _TEROW)
        def _pref():
            pltpu.async_copy(dst.at[pl.ds(prefetch_rb, _ECH_ROWS)],
                             dstv.at[1 - par], semi)

        for cp in ecps:
            cp.wait()
        for r in range(_ECH_ROWS):
            for j in range(8):
                v = dstv[par, r, pl.ds(j * 16, 16)]
                loc = v - cbase
                ok = (loc >= 0) & (loc < _HALF)
                oki = ok.astype(jnp.int32)
                pos = cnt + plsc.cumsum(oki) - 1
                plsc.store_scatter(psidx, [pos], loc, mask=ok)
                for f in range(_EF):
                    plsc.store_scatter(pf[f], [pos],
                                       eidxv[f, r, pl.ds(j * 16, 16)],
                                       mask=ok)
                cnt = cnt + jnp.sum(oki)
        pad8 = (-cnt) & 7
        padmask = il16 < pad8
        padpos = cnt + il16
        plsc.store_scatter(psidx, [padpos], _HALF + (il16 & 127), mask=padmask)
        for f in range(_EF):
            plsc.store_scatter(pf[f], [padpos], jnp.zeros((16,), jnp.int32),
                               mask=padmask)
        cnt = cnt + pad8

        @pl.when(cnt >= _FLUSH)
        def _fl():
            flush_window(cnt - _FLUSH)

        return jnp.where(cnt >= _FLUSH, cnt - _FLUSH, cnt)

    pltpu.async_copy(dst.at[pl.ds(ebase, _ECH_ROWS)], dstv.at[0], semi)

    def edge_pair(i, cnt):
        rb = ebase + i * 2 * _ECH_ROWS
        cnt = compact_and_flush(rb, 0, cnt, rb + _ECH_ROWS)
        cnt = compact_and_flush(rb + _ECH_ROWS, 1, cnt, rb + 2 * _ECH_ROWS)
        return cnt

    cnt = lax.fori_loop(0, _ECHUNKS // 2, edge_pair, jnp.int32(0))

    def pad16(t, cnt):
        padn = (-cnt) & 127

        @pl.when(t * 16 < padn)
        def _p():
            m = il16 < (padn - t * 16)
            ppos = cnt + t * 16 + il16
            plsc.store_scatter(psidx, [ppos], _HALF + (il16 & 127), mask=m)
            for f in range(_EF):
                plsc.store_scatter(pf[f], [ppos],
                                   jnp.zeros((16,), jnp.int32), mask=m)
        return cnt

    lax.fori_loop(0, 8, pad16, cnt)
    ngroups = (cnt + 127) // 128

    def drain_group(k, _):
        @pl.when(k < ngroups)
        def _dg():
            for m in range(8):
                sidx_dma[0, pl.ds(m * 16, 16)] = psidx[
                    pl.ds(pl.multiple_of(k * 128 + m * 16, 8), 16)]
            gd = pltpu.async_copy(
                eflat.at[pf[0].at[pl.ds(pl.multiple_of(k * 128, 8), 128)]],
                rows.at[0], semg)
            for f in range(_EF):
                gd.wait()
                if f + 1 < _EF:
                    gd = pltpu.async_copy(
                        eflat.at[pf[f + 1].at[
                            pl.ds(pl.multiple_of(k * 128, 8), 128)]],
                        rows.at[(f + 1) % 2], semg)
                pltpu.sync_copy(rows.at[f % 2], acc.at[sidx_dma.at[0]],
                                add=True)
        return _

    lax.fori_loop(0, 8, drain_group, 0)
    plsc.subcore_barrier()

    def out_chunk(k, _):
        cid = s + k * _NS

        @pl.when(cid < _NCHUNKS)
        def _do():
            pltpu.sync_copy(acc.at[pl.ds(cid * 1024, 1024)],
                            out.at[c, pl.ds(cid * 1024, 1024)])
        return _

    lax.fori_loop(0, (_NCHUNKS + _NS - 1) // _NS, out_chunk, 0)


def _mlp_body(a_ref, we_ref, wd_ref, o_ref):
    a = a_ref[0]
    h = jnp.maximum(jnp.dot(a, we_ref[...], preferred_element_type=jnp.float32), 0.0)
    o_ref[...] = jnp.dot(h, wd_ref[...], preferred_element_type=jnp.float32)


@jax.jit
def kernel(x, edge_attr, edge_index, node_tables, edge_tables, W_enc, W_dec):
    nflat = node_tables.reshape(_NF * _NV, _D)
    eflat = edge_tables.reshape(_EF * _EV, _D)

    nidx = x + (jnp.arange(_NF, dtype=jnp.int32) * _NV)[None, :]
    padn = jnp.zeros((_LROWS - _HALF, _NF), jnp.int32)
    nidx = jnp.concatenate([nidx[:_HALF], padn, nidx[_HALF:], padn], axis=0)
    nidx = nidx.reshape(_NC, _LROWS, _NF).transpose(0, 2, 1)
    nidx = nidx.reshape(_NC, _NF, _NROW, 128)

    eidx = edge_attr + (jnp.arange(_EF, dtype=jnp.int32) * _EV)[None, :]
    pade = jnp.zeros((_EPAD - _E, _EF), jnp.int32)
    eidx = jnp.concatenate([eidx, pade], axis=0).T.reshape(_EF, _EROW, 128)

    dst = jnp.concatenate([edge_index[1], jnp.full((_EPAD - _E,), -1, jnp.int32)])
    dst = dst.reshape(_EROW, 128)

    mesh = plsc.VectorSubcoreMesh(core_axis_name="c", subcore_axis_name="s",
                                  num_cores=_NC, num_subcores=_NS)
    agg = pl.kernel(
        _sc_body,
        out_type=jax.ShapeDtypeStruct((_NC, _LROWS, _D), jnp.float32),
        mesh=mesh,
        compiler_params=pltpu.CompilerParams(use_tc_tiling_on_sc=False,
                                             needs_layout_passes=False),
        scratch_types=[
            pltpu.VMEM_SHARED((_LROWS, _D), jnp.float32),
            pltpu.VMEM((8, 128), jnp.int32),
            pltpu.VMEM((_EF, _ECH_ROWS, 128), jnp.int32),
            pltpu.VMEM((2, _ECH_ROWS, 128), jnp.int32),
            pltpu.VMEM((_POOL,), jnp.int32),
            pltpu.VMEM((_POOL,), jnp.int32),
            pltpu.VMEM((_POOL,), jnp.int32),
            pltpu.VMEM((_POOL,), jnp.int32),
            pltpu.VMEM((_POOL,), jnp.int32),
            pltpu.VMEM((2, 128), jnp.int32),
            pltpu.VMEM((2, 128, _D), jnp.float32),
            pltpu.SemaphoreType.DMA,
            pltpu.SemaphoreType.DMA,
            pltpu.SemaphoreType.DMA,
            pltpu.SemaphoreType.DMA,
        ],
    )(nflat, eflat, nidx, eidx, dst)

    bm = 2000
    nb = _HALF // bm
    out = pl.pallas_call(
        _mlp_body,
        grid=(_NC, nb),
        in_specs=[
            pl.BlockSpec((1, bm, _D), lambda c, i: (c, i, 0)),
            pl.BlockSpec((_D, _H), lambda c, i: (0, 0)),
            pl.BlockSpec((_H, _D), lambda c, i: (0, 0)),
        ],
        out_specs=pl.BlockSpec((bm, _D), lambda c, i: (c * nb + i, 0)),
        out_shape=jax.ShapeDtypeStruct((_N, _D), jnp.float32),
    )(agg, W_enc, W_dec)
    return out
